# baseline (speedup 1.0000x reference)
import functools

import jax
import jax.numpy as jnp
from jax import lax
from jax.experimental import pallas as pl
from jax.experimental.pallas import tpu as pltpu
from jax.experimental.pallas import tpu_sc as plsc

MAX_LEN = 200
D = 64
DPAD = 128
BATCH = 4096
SEQ = 200

NC = 2
NS = 16
NW = NC * NS
BPW = BATCH // NW
NSLOT = 4


def _make_kernel():
    mesh = plsc.VectorSubcoreMesh(core_axis_name="c", subcore_axis_name="s")

    @functools.partial(
        pl.kernel,
        mesh=mesh,
        out_type=jax.ShapeDtypeStruct((BATCH, SEQ, DPAD), jnp.float32),
        scratch_types=[
            pltpu.VMEM((NSLOT, BPW), jnp.int32),
            pltpu.VMEM((NSLOT, BPW), jnp.int32),
            pltpu.VMEM((NSLOT, BPW, D), jnp.float32),
            pltpu.VMEM((MAX_LEN, D), jnp.float32),
        ] + [pltpu.SemaphoreType.DMA] * (3 * NSLOT),
        compiler_params=pltpu.CompilerParams(use_tc_tiling_on_sc=False),
    )
    def emb_kernel(xt_hbm, tok_hbm, pos_hbm, out_hbm, idx_v, idx2_v, rows_v,
                   pos_v, *sems):
        wid = lax.axis_index("s") * NC + lax.axis_index("c")
        b0 = wid * BPW
        isems = sems[0:NSLOT]
        gsems = sems[NSLOT:2 * NSLOT]
        osems = sems[2 * NSLOT:3 * NSLOT]

        pltpu.sync_copy(pos_hbm, pos_v)

        def idx_copy(l, slot):
            return pltpu.make_async_copy(
                xt_hbm.at[l, pl.ds(b0, BPW)], idx_v.at[slot], isems[slot]
            )

        def gather_copy(slot):
            return pltpu.make_async_copy(
                tok_hbm.at[idx2_v.at[slot]], rows_v.at[slot], gsems[slot]
            )

        def double_idx(slot):
            for j in range(BPW // 16):
                sl = pl.ds(j * 16, 16)
                idx2_v[slot, sl] = idx_v[slot, sl] * 2

        def out_copy(l, slot):
            return pltpu.make_async_copy(
                rows_v.at[slot],
                out_hbm.at[pl.ds(b0, BPW), l, pl.ds(0, D)],
                osems[slot],
            )

        def add_pos(l, slot):
            pvs = [pos_v[l, pl.ds(j * 16, 16)] for j in range(D // 16)]

            @plsc.parallel_loop(0, BPW, unroll=8)
            def _(t):
                for j in range(D // 16):
                    plsc.addupdate(rows_v.at[slot, t, pl.ds(j * 16, 16)], pvs[j])

        for s in range(NSLOT):
            idx_copy(s, s).start()
        idx_copy(0, 0).wait()
        double_idx(0)
        gather_copy(0).start()

        def step(l, slot):
            nxt = (slot + 1) % NSLOT
            gather_copy(slot).wait()

            @pl.when(l + NSLOT < SEQ)
            def _():
                idx_copy(l + NSLOT, slot).start()

            @pl.when(l + 1 < SEQ)
            def _():
                idx_copy(l + 1, nxt).wait()
                double_idx(nxt)
                @pl.when(l - (NSLOT - 1) >= 0)
                def _():
                    out_copy(l - (NSLOT - 1), nxt).wait()
                gather_copy(nxt).start()
            add_pos(l, slot)
            out_copy(l, slot).start()

        def body(g, carry):
            for u in range(NSLOT):
                step(g * NSLOT + u, u)
            return carry

        lax.fori_loop(0, SEQ // NSLOT, body, 0)

        for s in range(NSLOT):
            out_copy(SEQ - NSLOT + s, s).wait()

    return emb_kernel


_emb = _make_kernel()


@jax.jit
def kernel(x, token_table, pos_table):
    xt = x.astype(jnp.int32).T
    tpad = jnp.pad(token_table, ((0, 0), (0, DPAD - D))).reshape(-1, D)
    out = _emb(xt, tpad, pos_table)
    return out[:, :, :D]
